# Initial kernel scaffold; baseline (speedup 1.0000x reference)
import jax
import jax.numpy as jnp
from jax.experimental import pallas as pl


def kernel(x_in, f_w, f_b, p_w, p_b, edge_alpha, edge_beta):
    raise NotImplementedError("write your pallas kernel here")


# speedup vs baseline: 30.2979x; 30.2979x over previous
import jax
import jax.numpy as jnp
from jax.experimental import pallas as pl

_DIM = 384
_DF = 24
_WS = 7
_K = 9
_PATCH = 32
_N = _PATCH * _PATCH
_NP = _WS * _WS
_HW = 224 * 224
_CHUNK = 3584
_NCHUNK = _HW // _CHUNK


def _proj_kernel(x_ref, w_ref, b_ref, o_ref):
    o_ref[...] = (
        jnp.dot(w_ref[...], x_ref[...], preferred_element_type=jnp.float32)
        + b_ref[...]
    )


def _graph_kernel(fn_ref, ab_ref, o_ref):
    x = fn_ref[0]
    ab = ab_ref[...]
    alpha = ab[:, 0:1]
    beta = ab[:, 1:2]
    n2 = jnp.sum(x * x, axis=1, keepdims=True)
    inv = 1.0 / jnp.maximum(jnp.sqrt(n2), 1e-8)
    xn = x * inv
    sim = jax.lax.dot_general(
        xn, xn, (((1,), (1,)), ((), ())), preferred_element_type=jnp.float32
    )
    s = sim
    neg = jnp.float32(-jnp.inf)
    for _ in range(_K - 1):
        m = jnp.max(s, axis=1, keepdims=True)
        s = jnp.where(s >= m, neg, s)
    t = jnp.max(s, axis=1, keepdims=True)
    w = jnp.where(sim >= t, jax.nn.sigmoid(beta + alpha * sim), 0.0)
    ssum = jnp.sum(w, axis=1, keepdims=True)
    agg = jnp.dot(w, x, preferred_element_type=jnp.float32)
    o_ref[0] = agg / (ssum + 1e-12)


def kernel(x_in, f_w, f_b, p_w, p_b, edge_alpha, edge_beta):
    B, C, H, W = x_in.shape
    x2 = x_in.reshape(C, _HW)
    ab = jnp.concatenate([edge_alpha, edge_beta]).reshape(1, 2)

    f_pix = pl.pallas_call(
        _proj_kernel,
        grid=(_NCHUNK,),
        in_specs=[
            pl.BlockSpec((C, _CHUNK), lambda i: (0, i)),
            pl.BlockSpec((_DF, C), lambda i: (0, 0)),
            pl.BlockSpec((_DF, 1), lambda i: (0, 0)),
        ],
        out_specs=pl.BlockSpec((_DF, _CHUNK), lambda i: (0, i)),
        out_shape=jax.ShapeDtypeStruct((_DF, _HW), jnp.float32),
    )(x2, f_w, f_b.reshape(_DF, 1))

    f_nodes = (
        f_pix.reshape(_DF, _WS, _PATCH, _WS, _PATCH)
        .transpose(1, 3, 2, 4, 0)
        .reshape(_NP, _N, _DF)
    )

    out_nodes = pl.pallas_call(
        _graph_kernel,
        grid=(_NP,),
        in_specs=[
            pl.BlockSpec((1, _N, _DF), lambda p: (p, 0, 0)),
            pl.BlockSpec((1, 2), lambda p: (0, 0)),
        ],
        out_specs=pl.BlockSpec((1, _N, _DF), lambda p: (p, 0, 0)),
        out_shape=jax.ShapeDtypeStruct((_NP, _N, _DF), jnp.float32),
    )(f_nodes, ab)

    out_pix = (
        out_nodes.reshape(_WS, _WS, _PATCH, _PATCH, _DF)
        .transpose(4, 0, 2, 1, 3)
        .reshape(_DF, _HW)
    )

    out = pl.pallas_call(
        _proj_kernel,
        grid=(_NCHUNK,),
        in_specs=[
            pl.BlockSpec((_DF, _CHUNK), lambda i: (0, i)),
            pl.BlockSpec((C, _DF), lambda i: (0, 0)),
            pl.BlockSpec((C, 1), lambda i: (0, 0)),
        ],
        out_specs=pl.BlockSpec((C, _CHUNK), lambda i: (0, i)),
        out_shape=jax.ShapeDtypeStruct((C, _HW), jnp.float32),
    )(out_pix, p_w, p_b.reshape(C, 1))

    return out.reshape(B, C, _HW)


# speedup vs baseline: 31.2508x; 1.0314x over previous
import jax
import jax.numpy as jnp
from jax.experimental import pallas as pl

_DIM = 384
_DF = 24
_WS = 7
_K = 9
_PATCH = 32
_N = _PATCH * _PATCH
_NP = _WS * _WS
_HW = 224 * 224
_CHUNK = 3584
_NCHUNK = _HW // _CHUNK


def _proj_kernel(x_ref, w_ref, b_ref, o_ref):
    o_ref[...] = (
        jnp.dot(w_ref[...], x_ref[...], preferred_element_type=jnp.float32)
        + b_ref[...]
    )


def _graph_kernel(fn_ref, ab_ref, o_ref):
    x = fn_ref[0]
    ab = ab_ref[...]
    alpha = ab[:, 0:1]
    beta = ab[:, 1:2]
    n2 = jnp.sum(x * x, axis=1, keepdims=True)
    inv = 1.0 / jnp.maximum(jnp.sqrt(n2), 1e-8)
    xn = x * inv
    sim = jax.lax.dot_general(
        xn, xn, (((1,), (1,)), ((), ())), preferred_element_type=jnp.float32
    )
    neg = jnp.float32(-jnp.inf)
    m = jnp.max(sim, axis=1, keepdims=True)
    for _ in range(_K - 1):
        m = jnp.max(jnp.where(sim < m, sim, neg), axis=1, keepdims=True)
    w = jnp.where(sim >= m, jax.nn.sigmoid(beta + alpha * sim), 0.0)
    xe = jnp.concatenate([x, jnp.ones((_N, 1), jnp.float32)], axis=1)
    agg = jnp.dot(w, xe, preferred_element_type=jnp.float32)
    o_ref[0] = agg[:, :_DF] / (agg[:, _DF:_DF + 1] + 1e-12)


def kernel(x_in, f_w, f_b, p_w, p_b, edge_alpha, edge_beta):
    B, C, H, W = x_in.shape
    x2 = x_in.reshape(C, _HW)
    ab = jnp.concatenate([edge_alpha, edge_beta]).reshape(1, 2)

    f_pix = pl.pallas_call(
        _proj_kernel,
        grid=(_NCHUNK,),
        in_specs=[
            pl.BlockSpec((C, _CHUNK), lambda i: (0, i)),
            pl.BlockSpec((_DF, C), lambda i: (0, 0)),
            pl.BlockSpec((_DF, 1), lambda i: (0, 0)),
        ],
        out_specs=pl.BlockSpec((_DF, _CHUNK), lambda i: (0, i)),
        out_shape=jax.ShapeDtypeStruct((_DF, _HW), jnp.float32),
    )(x2, f_w, f_b.reshape(_DF, 1))

    f_nodes = (
        f_pix.reshape(_DF, _WS, _PATCH, _WS, _PATCH)
        .transpose(1, 3, 2, 4, 0)
        .reshape(_NP, _N, _DF)
    )

    out_nodes = pl.pallas_call(
        _graph_kernel,
        grid=(_NP,),
        in_specs=[
            pl.BlockSpec((1, _N, _DF), lambda p: (p, 0, 0)),
            pl.BlockSpec((1, 2), lambda p: (0, 0)),
        ],
        out_specs=pl.BlockSpec((1, _N, _DF), lambda p: (p, 0, 0)),
        out_shape=jax.ShapeDtypeStruct((_NP, _N, _DF), jnp.float32),
    )(f_nodes, ab)

    out_pix = (
        out_nodes.reshape(_WS, _WS, _PATCH, _PATCH, _DF)
        .transpose(4, 0, 2, 1, 3)
        .reshape(_DF, _HW)
    )

    out = pl.pallas_call(
        _proj_kernel,
        grid=(_NCHUNK,),
        in_specs=[
            pl.BlockSpec((_DF, _CHUNK), lambda i: (0, i)),
            pl.BlockSpec((C, _DF), lambda i: (0, 0)),
            pl.BlockSpec((C, 1), lambda i: (0, 0)),
        ],
        out_specs=pl.BlockSpec((C, _CHUNK), lambda i: (0, i)),
        out_shape=jax.ShapeDtypeStruct((C, _HW), jnp.float32),
    )(out_pix, p_w, p_b.reshape(C, 1))

    return out.reshape(B, C, _HW)
